# Initial kernel scaffold; baseline (speedup 1.0000x reference)
import jax
import jax.numpy as jnp
from jax.experimental import pallas as pl


def kernel(x, edge_index, graph_indices, W1s, W1n, b1, W2s, W2n, b2, mWs, mWn, mb, madvW, madvb, mvalW, mvalb, bWs, bWn, bb, badvW, badvb, bvalW, bvalb):
    raise NotImplementedError("write your pallas kernel here")


# speedup vs baseline: 7.1330x; 7.1330x over previous
import jax
import jax.numpy as jnp
from jax import lax
from jax.experimental import pallas as pl
from jax.experimental.pallas import tpu as pltpu
from jax.experimental.pallas import tpu_sc as plsc

NSC = 2
NTILE = 16
CH = 128
G = 14
ZB = 50
BN = 2000


def _sc_segsum(d, n, nchunks, core_indexed_table):
    na = n + 400
    assert na % (NTILE * ZB) == 0 and n % (NTILE * 625) == 0
    ngroups = nchunks // G
    ptg = ngroups // (NTILE if core_indexed_table else NTILE * NSC)
    slabs = n // 625

    def body(tbl_ref, src_ref, dst_ref, out_ref, acc, sidx, didx, rows, zbuf,
             sem):
        c = lax.axis_index("c")
        s = lax.axis_index("s")

        def zrow(i, _):
            for j in range(d // 16):
                zbuf[i, pl.ds(16 * j, 16)] = jnp.zeros((16,), jnp.float32)
            return 0
        lax.fori_loop(0, ZB, zrow, 0)
        zbase = s * (na // NTILE)

        def zcp(i, _):
            pltpu.sync_copy(zbuf, acc.at[pl.ds(zbase + i * ZB, ZB)])
            return 0
        lax.fori_loop(0, na // (NTILE * ZB), zcp, 0)
        plsc.subcore_barrier()

        if core_indexed_table:
            tbl = tbl_ref.at[c]
            gbase = s * ptg
        else:
            tbl = tbl_ref.at[0]
            gbase = (c * NTILE + s) * ptg

        def group(g, _):
            pltpu.sync_copy(src_ref.at[gbase + g], sidx)
            pltpu.sync_copy(dst_ref.at[gbase + g], didx)

            def one(j, _):
                pltpu.async_copy(tbl.at[sidx.at[j]], rows, sem).wait()
                pltpu.sync_copy(rows, acc.at[didx.at[j]], add=True)
                return 0
            lax.fori_loop(0, G, one, 0)
            return 0
        lax.fori_loop(0, ptg, group, 0)
        plsc.subcore_barrier()

        spt = slabs // NTILE

        def ocp(i, _):
            pltpu.sync_copy(acc.at[pl.ds((s * spt + i) * 625, 625)],
                            out_ref.at[c, s * spt + i])
            return 0
        lax.fori_loop(0, spt, ocp, 0)

    mesh = plsc.VectorSubcoreMesh(core_axis_name="c", subcore_axis_name="s")
    return pl.kernel(
        body,
        out_type=jax.ShapeDtypeStruct((NSC, slabs, 625, d), jnp.float32),
        mesh=mesh,
        compiler_params=pltpu.CompilerParams(use_tc_tiling_on_sc=False),
        scratch_types=[
            pltpu.VMEM_SHARED((na, d), jnp.float32),
            pltpu.VMEM((G, CH), jnp.int32),
            pltpu.VMEM((G, CH), jnp.int32),
            pltpu.VMEM((CH, d), jnp.float32),
            pltpu.VMEM((ZB, d), jnp.float32),
            pltpu.SemaphoreType.DMA,
        ],
    )


def _dot(a, b):
    return jnp.dot(a, b, preferred_element_type=jnp.float32)


def _layer1_body(p_ref, x_ref, w_ref, b_ref, h_ref, inv_ref):
    p = p_ref[0] + p_ref[1]
    inv = 1.0 / jnp.maximum(p[:, 2:3], 1.0)
    inp = jnp.concatenate([x_ref[:, 0:2], p[:, 0:2] * inv], axis=1)
    h = jnp.maximum(_dot(inp, w_ref[...]) + b_ref[...], 0.0)
    h_ref[0] = h[:, 0:32]
    h_ref[1] = h[:, 32:64]
    inv_ref[...] = inv


def _layer_body(h_ref, s_ref, inv_ref, w_ref, b_ref, o_ref):
    h = jnp.concatenate([h_ref[0], h_ref[1]], axis=1)
    agg = jnp.concatenate([s_ref[0], s_ref[1]], axis=1) * inv_ref[...]
    o = jnp.maximum(_dot(jnp.concatenate([h, agg], axis=1), w_ref[...])
                    + b_ref[...], 0.0)
    o_ref[0] = o[:, 0:32]
    o_ref[1] = o[:, 32:64]


def _head_body(h_ref, s_ref, inv_ref, w_ref, b_ref, aw_ref, ab_ref, gi_ref,
               adv_ref, ps_ref, as_ref, cnt_ref):
    nb = gi_ref.shape[0]
    h = jnp.concatenate([h_ref[0], h_ref[1]], axis=1)
    agg = jnp.concatenate([s_ref[0], s_ref[1]], axis=1) * inv_ref[...]
    hh = jnp.maximum(_dot(jnp.concatenate([h, agg], axis=1), w_ref[...])
                     + b_ref[...], 0.0)
    advn = 2.0 * jnp.tanh(_dot(hh, aw_ref[...]) + ab_ref[...])
    adv_ref[...] = advn

    nseg = ps_ref.shape[0]
    ids = lax.broadcasted_iota(jnp.int32, (1, nseg), 1)
    oh = (gi_ref[...] == ids).astype(jnp.float32)

    @pl.when(pl.program_id(0) == 0)
    def _():
        ps_ref[...] = jnp.zeros_like(ps_ref)
        as_ref[...] = jnp.zeros_like(as_ref)
        cnt_ref[...] = jnp.zeros_like(cnt_ref)

    dn = (((0,), (0,)), ((), ()))
    ps_ref[...] += lax.dot_general(oh, hh, dn,
                                   preferred_element_type=jnp.float32)
    as_ref[...] += lax.dot_general(oh, advn, dn,
                                   preferred_element_type=jnp.float32)
    cnt_ref[...] += lax.dot_general(oh, jnp.ones((nb, 1), jnp.float32), dn,
                                    preferred_element_type=jnp.float32)


def _combine_body(ps_ref, cnt_ref, as_ref, vw_ref, vb_ref, adv_ref, gi_ref,
                  out_ref):
    nseg = ps_ref.shape[0]
    cnt = jnp.maximum(cnt_ref[...], 1.0)
    pooled = ps_ref[...] / cnt
    value = jnp.tanh(_dot(pooled, vw_ref[...]) + vb_ref[...])
    combined = value - as_ref[...] / cnt
    ids = lax.broadcasted_iota(jnp.int32, (1, nseg), 1)
    oh = (gi_ref[...] == ids).astype(jnp.float32)
    out_ref[...] = _dot(oh, combined) + adv_ref[...]


def kernel(x, edge_index, graph_indices, W1s, W1n, b1, W2s, W2n, b2, mWs, mWn,
           mb, madvW, madvb, mvalW, mvalb, bWs, bWn, bb, badvW, badvb, bvalW,
           bvalb):
    n = x.shape[0]
    e = edge_index.shape[1]
    nseg = 256
    f32 = jnp.float32

    is_maker = x[0, 2] == 1.0
    W3s = jnp.where(is_maker, mWs, bWs)
    W3n = jnp.where(is_maker, mWn, bWn)
    b3 = jnp.where(is_maker, mb, bb)
    advW = jnp.where(is_maker, madvW, badvW)
    advb = jnp.where(is_maker, madvb, badvb)
    valW = jnp.where(is_maker, mvalW, bvalW)
    valb = jnp.where(is_maker, mvalb, bvalb)

    x16 = jnp.concatenate(
        [x[:, 0:2], jnp.ones((n, 1), f32), jnp.zeros((n, 13), f32)], axis=1)

    step = NSC * NTILE * G * CH
    epad = ((e + step - 1) // step) * step
    trash = n + (jnp.arange(epad - e, dtype=jnp.int32) % 256)
    src = jnp.concatenate(
        [edge_index[0], jnp.zeros((epad - e,), jnp.int32)]).reshape(-1, G, CH)
    dst = jnp.concatenate([edge_index[1], trash]).reshape(-1, G, CH)
    nchunks = epad // CH

    w1 = jnp.concatenate([W1s, W1n], axis=0)
    w2 = jnp.concatenate([W2s, W2n], axis=0)
    w3 = jnp.concatenate([W3s, W3n], axis=0)
    b1r, b2r, b3r = b1.reshape(1, -1), b2.reshape(1, -1), b3.reshape(1, -1)
    advbr, valbr = advb.reshape(1, 1), valb.reshape(1, 1)
    gi2 = graph_indices.reshape(n, 1)

    seg1 = _sc_segsum(16, n, nchunks, core_indexed_table=False)
    seg2 = _sc_segsum(32, n, nchunks, core_indexed_table=True)

    grid = (n // BN,)
    full2 = lambda shp: pl.BlockSpec(shp, lambda i: (0, 0))
    rows2 = lambda d: pl.BlockSpec((BN, d), lambda i: (i, 0))
    rows3 = lambda d: pl.BlockSpec((NSC, BN, d), lambda i: (0, i, 0))

    p1 = seg1(x16.reshape(1, n, 16), src, dst).reshape(NSC, n, 16)
    h1, inv = pl.pallas_call(
        _layer1_body,
        grid=grid,
        in_specs=[rows3(16), rows2(16), full2((4, 64)), full2((1, 64))],
        out_specs=[rows3(32), rows2(1)],
        out_shape=[jax.ShapeDtypeStruct((NSC, n, 32), f32),
                   jax.ShapeDtypeStruct((n, 1), f32)],
    )(p1, x16, w1, b1r)

    s2 = seg2(h1, src, dst).reshape(NSC, n, 32)
    h2 = pl.pallas_call(
        _layer_body,
        grid=grid,
        in_specs=[rows3(32), rows3(32), rows2(1), full2((128, 64)),
                  full2((1, 64))],
        out_specs=rows3(32),
        out_shape=jax.ShapeDtypeStruct((NSC, n, 32), f32),
    )(h1, s2, inv, w2, b2r)

    s3 = seg2(h2, src, dst).reshape(NSC, n, 32)
    advn, psum, asum, cnt = pl.pallas_call(
        _head_body,
        grid=grid,
        in_specs=[rows3(32), rows3(32), rows2(1), full2((128, 64)),
                  full2((1, 64)), full2((64, 1)), full2((1, 1)), rows2(1)],
        out_specs=[rows2(1), full2((nseg, 64)), full2((nseg, 1)),
                   full2((nseg, 1))],
        out_shape=[jax.ShapeDtypeStruct((n, 1), f32),
                   jax.ShapeDtypeStruct((nseg, 64), f32),
                   jax.ShapeDtypeStruct((nseg, 1), f32),
                   jax.ShapeDtypeStruct((nseg, 1), f32)],
    )(h2, s3, inv, w3, b3r, advW, advbr, gi2)

    out = pl.pallas_call(
        _combine_body,
        grid=grid,
        in_specs=[full2((nseg, 64)), full2((nseg, 1)), full2((nseg, 1)),
                  full2((64, 1)), full2((1, 1)), rows2(1), rows2(1)],
        out_specs=rows2(1),
        out_shape=jax.ShapeDtypeStruct((n, 1), f32),
    )(psum, cnt, asum, valW, valbr, advn, gi2)
    return out[:, 0]


# speedup vs baseline: 9.7260x; 1.3635x over previous
import jax
import jax.numpy as jnp
from jax import lax
from jax.experimental import pallas as pl
from jax.experimental.pallas import tpu as pltpu
from jax.experimental.pallas import tpu_sc as plsc

NSC = 2
NTILE = 16
CH = 128
G = 7
SLAB = 49
ZB = 50
BN = 2000


def _sc_segsum(d, n, nchunks, core_indexed_table):
    na = n + 400
    assert na % (NTILE * ZB) == 0 and n % (NTILE * 625) == 0
    ptc = nchunks // (NTILE if core_indexed_table else NTILE * NSC)
    ptg = ptc // G
    slabs = n // 625

    def body(tbl_ref, src_ref, dst_ref, out_ref, acc, sidx, didx, rows, zbuf,
             gsem0, gsem1, ssem0, ssem1):
        gsem = (gsem0, gsem1)
        ssem = (ssem0, ssem1)
        c = lax.axis_index("c")
        s = lax.axis_index("s")

        def zrow(i, _):
            for j in range(d // 16):
                zbuf[i, pl.ds(16 * j, 16)] = jnp.zeros((16,), jnp.float32)
            return 0
        lax.fori_loop(0, ZB, zrow, 0)
        zbase = s * (na // NTILE)

        def zcp(i, _):
            pltpu.sync_copy(zbuf, acc.at[pl.ds(zbase + i * ZB, ZB)])
            return 0
        lax.fori_loop(0, na // (NTILE * ZB), zcp, 0)
        plsc.subcore_barrier()

        if core_indexed_table:
            tbl = tbl_ref.at[c]
            tid = s
        else:
            tbl = tbl_ref.at[0]
            tid = c * NTILE + s

        def group(g, _):
            b = g * G
            gd = {0: pltpu.async_copy(tbl.at[sidx.at[b]], rows.at[0],
                                      gsem[0])}
            sd = {}
            for j in range(G):
                p = j % 2
                if j + 1 < G:
                    if j - 1 in sd:
                        sd.pop(j - 1).wait()
                    gd[j + 1] = pltpu.async_copy(tbl.at[sidx.at[b + j + 1]],
                                                 rows.at[1 - p], gsem[1 - p])
                gd.pop(j).wait()
                sd[j] = pltpu.async_copy(rows.at[p], acc.at[didx.at[b + j]],
                                         ssem[p], add=True)
            for j in sorted(sd):
                sd.pop(j).wait()
            return 0

        def slab(i, _):
            pltpu.sync_copy(src_ref.at[tid, i], sidx)
            pltpu.sync_copy(dst_ref.at[tid, i], didx)
            lax.fori_loop(0, SLAB // G, group, 0)
            return 0
        lax.fori_loop(0, ptc // SLAB, slab, 0)
        plsc.subcore_barrier()

        spt = slabs // NTILE

        def ocp(i, _):
            pltpu.sync_copy(acc.at[pl.ds((s * spt + i) * 625, 625)],
                            out_ref.at[c, s * spt + i])
            return 0
        lax.fori_loop(0, spt, ocp, 0)

    mesh = plsc.VectorSubcoreMesh(core_axis_name="c", subcore_axis_name="s")
    return pl.kernel(
        body,
        out_type=jax.ShapeDtypeStruct((NSC, slabs, 625, d), jnp.float32),
        mesh=mesh,
        compiler_params=pltpu.CompilerParams(use_tc_tiling_on_sc=False),
        scratch_types=[
            pltpu.VMEM_SHARED((na, d), jnp.float32),
            pltpu.VMEM((SLAB, CH), jnp.int32),
            pltpu.VMEM((SLAB, CH), jnp.int32),
            pltpu.VMEM((2, CH, d), jnp.float32),
            pltpu.VMEM((ZB, d), jnp.float32),
            pltpu.SemaphoreType.DMA,
            pltpu.SemaphoreType.DMA,
            pltpu.SemaphoreType.DMA,
            pltpu.SemaphoreType.DMA,
        ],
    )


def _dot(a, b):
    return jnp.dot(a, b, preferred_element_type=jnp.float32)


def _layer1_body(p_ref, x_ref, w_ref, b_ref, h_ref, inv_ref):
    p = p_ref[0] + p_ref[1]
    inv = 1.0 / jnp.maximum(p[:, 2:3], 1.0)
    inp = jnp.concatenate([x_ref[:, 0:2], p[:, 0:2] * inv], axis=1)
    h = jnp.maximum(_dot(inp, w_ref[...]) + b_ref[...], 0.0)
    h_ref[0] = h[:, 0:32]
    h_ref[1] = h[:, 32:64]
    inv_ref[...] = inv


def _layer_body(h_ref, s_ref, inv_ref, w_ref, b_ref, o_ref):
    h = jnp.concatenate([h_ref[0], h_ref[1]], axis=1)
    agg = jnp.concatenate([s_ref[0], s_ref[1]], axis=1) * inv_ref[...]
    o = jnp.maximum(_dot(jnp.concatenate([h, agg], axis=1), w_ref[...])
                    + b_ref[...], 0.0)
    o_ref[0] = o[:, 0:32]
    o_ref[1] = o[:, 32:64]


def _head_body(h_ref, s_ref, inv_ref, w_ref, b_ref, aw_ref, ab_ref, gi_ref,
               adv_ref, ps_ref, as_ref, cnt_ref):
    nb = gi_ref.shape[0]
    h = jnp.concatenate([h_ref[0], h_ref[1]], axis=1)
    agg = jnp.concatenate([s_ref[0], s_ref[1]], axis=1) * inv_ref[...]
    hh = jnp.maximum(_dot(jnp.concatenate([h, agg], axis=1), w_ref[...])
                     + b_ref[...], 0.0)
    advn = 2.0 * jnp.tanh(_dot(hh, aw_ref[...]) + ab_ref[...])
    adv_ref[...] = advn

    nseg = ps_ref.shape[0]
    ids = lax.broadcasted_iota(jnp.int32, (1, nseg), 1)
    oh = (gi_ref[...] == ids).astype(jnp.float32)

    @pl.when(pl.program_id(0) == 0)
    def _():
        ps_ref[...] = jnp.zeros_like(ps_ref)
        as_ref[...] = jnp.zeros_like(as_ref)
        cnt_ref[...] = jnp.zeros_like(cnt_ref)

    dn = (((0,), (0,)), ((), ()))
    ps_ref[...] += lax.dot_general(oh, hh, dn,
                                   preferred_element_type=jnp.float32)
    as_ref[...] += lax.dot_general(oh, advn, dn,
                                   preferred_element_type=jnp.float32)
    cnt_ref[...] += lax.dot_general(oh, jnp.ones((nb, 1), jnp.float32), dn,
                                    preferred_element_type=jnp.float32)


def _combine_body(ps_ref, cnt_ref, as_ref, vw_ref, vb_ref, adv_ref, gi_ref,
                  out_ref):
    nseg = ps_ref.shape[0]
    cnt = jnp.maximum(cnt_ref[...], 1.0)
    pooled = ps_ref[...] / cnt
    value = jnp.tanh(_dot(pooled, vw_ref[...]) + vb_ref[...])
    combined = value - as_ref[...] / cnt
    ids = lax.broadcasted_iota(jnp.int32, (1, nseg), 1)
    oh = (gi_ref[...] == ids).astype(jnp.float32)
    out_ref[...] = _dot(oh, combined) + adv_ref[...]


def kernel(x, edge_index, graph_indices, W1s, W1n, b1, W2s, W2n, b2, mWs, mWn,
           mb, madvW, madvb, mvalW, mvalb, bWs, bWn, bb, badvW, badvb, bvalW,
           bvalb):
    n = x.shape[0]
    e = edge_index.shape[1]
    nseg = 256
    f32 = jnp.float32

    is_maker = x[0, 2] == 1.0
    W3s = jnp.where(is_maker, mWs, bWs)
    W3n = jnp.where(is_maker, mWn, bWn)
    b3 = jnp.where(is_maker, mb, bb)
    advW = jnp.where(is_maker, madvW, badvW)
    advb = jnp.where(is_maker, madvb, badvb)
    valW = jnp.where(is_maker, mvalW, bvalW)
    valb = jnp.where(is_maker, mvalb, bvalb)

    x16 = jnp.concatenate(
        [x[:, 0:2], jnp.ones((n, 1), f32), jnp.zeros((n, 13), f32)], axis=1)

    step = NSC * NTILE * G * CH
    epad = ((e + step - 1) // step) * step
    trash = n + (jnp.arange(epad - e, dtype=jnp.int32) % 256)
    src = jnp.concatenate(
        [edge_index[0], jnp.zeros((epad - e,), jnp.int32)]).reshape(-1, CH)
    dst = jnp.concatenate([edge_index[1], trash]).reshape(-1, CH)
    nchunks = epad // CH

    w1 = jnp.concatenate([W1s, W1n], axis=0)
    w2 = jnp.concatenate([W2s, W2n], axis=0)
    w3 = jnp.concatenate([W3s, W3n], axis=0)
    b1r, b2r, b3r = b1.reshape(1, -1), b2.reshape(1, -1), b3.reshape(1, -1)
    advbr, valbr = advb.reshape(1, 1), valb.reshape(1, 1)
    gi2 = graph_indices.reshape(n, 1)

    seg1 = _sc_segsum(16, n, nchunks, core_indexed_table=False)
    seg2 = _sc_segsum(32, n, nchunks, core_indexed_table=True)

    grid = (n // BN,)
    full2 = lambda shp: pl.BlockSpec(shp, lambda i: (0, 0))
    rows2 = lambda d: pl.BlockSpec((BN, d), lambda i: (i, 0))
    rows3 = lambda d: pl.BlockSpec((NSC, BN, d), lambda i: (0, i, 0))

    src1 = src.reshape(NSC * NTILE, -1, SLAB, CH)
    dst1 = dst.reshape(NSC * NTILE, -1, SLAB, CH)
    src2 = src.reshape(NTILE, -1, SLAB, CH)
    dst2 = dst.reshape(NTILE, -1, SLAB, CH)

    p1 = seg1(x16.reshape(1, n, 16), src1, dst1).reshape(NSC, n, 16)
    h1, inv = pl.pallas_call(
        _layer1_body,
        grid=grid,
        in_specs=[rows3(16), rows2(16), full2((4, 64)), full2((1, 64))],
        out_specs=[rows3(32), rows2(1)],
        out_shape=[jax.ShapeDtypeStruct((NSC, n, 32), f32),
                   jax.ShapeDtypeStruct((n, 1), f32)],
    )(p1, x16, w1, b1r)

    s2 = seg2(h1, src2, dst2).reshape(NSC, n, 32)
    h2 = pl.pallas_call(
        _layer_body,
        grid=grid,
        in_specs=[rows3(32), rows3(32), rows2(1), full2((128, 64)),
                  full2((1, 64))],
        out_specs=rows3(32),
        out_shape=jax.ShapeDtypeStruct((NSC, n, 32), f32),
    )(h1, s2, inv, w2, b2r)

    s3 = seg2(h2, src2, dst2).reshape(NSC, n, 32)
    advn, psum, asum, cnt = pl.pallas_call(
        _head_body,
        grid=grid,
        in_specs=[rows3(32), rows3(32), rows2(1), full2((128, 64)),
                  full2((1, 64)), full2((64, 1)), full2((1, 1)), rows2(1)],
        out_specs=[rows2(1), full2((nseg, 64)), full2((nseg, 1)),
                   full2((nseg, 1))],
        out_shape=[jax.ShapeDtypeStruct((n, 1), f32),
                   jax.ShapeDtypeStruct((nseg, 64), f32),
                   jax.ShapeDtypeStruct((nseg, 1), f32),
                   jax.ShapeDtypeStruct((nseg, 1), f32)],
    )(h2, s3, inv, w3, b3r, advW, advbr, gi2)

    out = pl.pallas_call(
        _combine_body,
        grid=grid,
        in_specs=[full2((nseg, 64)), full2((nseg, 1)), full2((nseg, 1)),
                  full2((64, 1)), full2((1, 1)), rows2(1), rows2(1)],
        out_specs=rows2(1),
        out_shape=jax.ShapeDtypeStruct((n, 1), f32),
    )(psum, cnt, asum, valW, valbr, advn, gi2)
    return out[:, 0]


# speedup vs baseline: 10.4475x; 1.0742x over previous
import jax
import jax.numpy as jnp
from jax import lax
from jax.experimental import pallas as pl
from jax.experimental.pallas import tpu as pltpu
from jax.experimental.pallas import tpu_sc as plsc

NSC = 2
NTILE = 16
CH = 128
G = 14
SLAB = 28
ZB = 150
BN = 2000


def _sc_segsum(d, n, nchunks, core_indexed_table):
    na = n + 400
    assert na % (NTILE * ZB) == 0 and n % (NTILE * 625) == 0
    ptc = nchunks // (NTILE if core_indexed_table else NTILE * NSC)
    ptg = ptc // G
    slabs = n // 625

    nsl = ptc // SLAB

    def body(tbl_ref, src_ref, dst_ref, out_ref, acc, sidx, didx, rows, zbuf,
             gsem0, gsem1, ssem0, ssem1, isem):
        gsem = (gsem0, gsem1)
        ssem = (ssem0, ssem1)
        c = lax.axis_index("c")
        s = lax.axis_index("s")

        if core_indexed_table:
            tbl = tbl_ref.at[c]
            tid = s
        else:
            tbl = tbl_ref.at[0]
            tid = c * NTILE + s

        pltpu.async_copy(src_ref.at[tid, 0], sidx.at[0], isem)
        pltpu.async_copy(dst_ref.at[tid, 0], didx.at[0], isem)

        def zrow(i, _):
            for j in range(d // 16):
                zbuf[i, pl.ds(16 * j, 16)] = jnp.zeros((16,), jnp.float32)
            return 0
        lax.fori_loop(0, ZB, zrow, 0)
        zbase = s * (na // NTILE)

        def zcp(i, _):
            pltpu.sync_copy(zbuf, acc.at[pl.ds(zbase + i * ZB, ZB)])
            return 0
        lax.fori_loop(0, na // (NTILE * ZB), zcp, 0)
        plsc.subcore_barrier()

        def slab(i, _):
            p = i % 2
            pltpu.make_async_copy(src_ref.at[tid, i], sidx.at[p], isem).wait()
            pltpu.make_async_copy(dst_ref.at[tid, i], didx.at[p], isem).wait()

            @pl.when(i + 1 < nsl)
            def _():
                pltpu.async_copy(src_ref.at[tid, i + 1], sidx.at[1 - p], isem)
                pltpu.async_copy(dst_ref.at[tid, i + 1], didx.at[1 - p], isem)

            def group(g, _):
                b = g * G
                gd = {0: pltpu.async_copy(tbl.at[sidx.at[p, b]], rows.at[0],
                                          gsem[0])}
                sd = {}
                for j in range(G):
                    q = j % 2
                    if j + 1 < G:
                        if j - 1 in sd:
                            sd.pop(j - 1).wait()
                        gd[j + 1] = pltpu.async_copy(
                            tbl.at[sidx.at[p, b + j + 1]], rows.at[1 - q],
                            gsem[1 - q])
                    gd.pop(j).wait()
                    sd[j] = pltpu.async_copy(rows.at[q],
                                             acc.at[didx.at[p, b + j]],
                                             ssem[q], add=True)
                for j in sorted(sd):
                    sd.pop(j).wait()
                return 0
            lax.fori_loop(0, SLAB // G, group, 0)
            return 0
        lax.fori_loop(0, nsl, slab, 0)
        plsc.subcore_barrier()

        spt = slabs // NTILE

        def ocp(i, _):
            pltpu.sync_copy(acc.at[pl.ds((s * spt + i) * 625, 625)],
                            out_ref.at[c, s * spt + i])
            return 0
        lax.fori_loop(0, spt, ocp, 0)

    mesh = plsc.VectorSubcoreMesh(core_axis_name="c", subcore_axis_name="s")
    return pl.kernel(
        body,
        out_type=jax.ShapeDtypeStruct((NSC, slabs, 625, d), jnp.float32),
        mesh=mesh,
        compiler_params=pltpu.CompilerParams(use_tc_tiling_on_sc=False),
        scratch_types=[
            pltpu.VMEM_SHARED((na, d), jnp.float32),
            pltpu.VMEM((2, SLAB, CH), jnp.int32),
            pltpu.VMEM((2, SLAB, CH), jnp.int32),
            pltpu.VMEM((2, CH, d), jnp.float32),
            pltpu.VMEM((ZB, d), jnp.float32),
            pltpu.SemaphoreType.DMA,
            pltpu.SemaphoreType.DMA,
            pltpu.SemaphoreType.DMA,
            pltpu.SemaphoreType.DMA,
            pltpu.SemaphoreType.DMA,
        ],
    )


def _dot(a, b):
    return jnp.dot(a, b, preferred_element_type=jnp.float32)


def _layer1_body(p_ref, x_ref, w_ref, b_ref, h_ref, inv_ref):
    p = p_ref[0] + p_ref[1]
    inv = 1.0 / jnp.maximum(p[:, 2:3], 1.0)
    inp = jnp.concatenate([x_ref[:, 0:2], p[:, 0:2] * inv], axis=1)
    h = jnp.maximum(_dot(inp, w_ref[...]) + b_ref[...], 0.0)
    h_ref[0] = h[:, 0:32]
    h_ref[1] = h[:, 32:64]
    inv_ref[...] = inv


def _layer_body(h_ref, s_ref, inv_ref, w_ref, b_ref, o_ref):
    h = jnp.concatenate([h_ref[0], h_ref[1]], axis=1)
    agg = jnp.concatenate([s_ref[0], s_ref[1]], axis=1) * inv_ref[...]
    o = jnp.maximum(_dot(jnp.concatenate([h, agg], axis=1), w_ref[...])
                    + b_ref[...], 0.0)
    o_ref[0] = o[:, 0:32]
    o_ref[1] = o[:, 32:64]


def _head_body(h_ref, s_ref, inv_ref, w_ref, b_ref, aw_ref, ab_ref, gi_ref,
               adv_ref, ps_ref, as_ref, cnt_ref):
    nb = gi_ref.shape[0]
    h = jnp.concatenate([h_ref[0], h_ref[1]], axis=1)
    agg = jnp.concatenate([s_ref[0], s_ref[1]], axis=1) * inv_ref[...]
    hh = jnp.maximum(_dot(jnp.concatenate([h, agg], axis=1), w_ref[...])
                     + b_ref[...], 0.0)
    advn = 2.0 * jnp.tanh(_dot(hh, aw_ref[...]) + ab_ref[...])
    adv_ref[...] = advn

    nseg = ps_ref.shape[0]
    ids = lax.broadcasted_iota(jnp.int32, (1, nseg), 1)
    oh = (gi_ref[...] == ids).astype(jnp.float32)

    @pl.when(pl.program_id(0) == 0)
    def _():
        ps_ref[...] = jnp.zeros_like(ps_ref)
        as_ref[...] = jnp.zeros_like(as_ref)
        cnt_ref[...] = jnp.zeros_like(cnt_ref)

    dn = (((0,), (0,)), ((), ()))
    ps_ref[...] += lax.dot_general(oh, hh, dn,
                                   preferred_element_type=jnp.float32)
    as_ref[...] += lax.dot_general(oh, advn, dn,
                                   preferred_element_type=jnp.float32)
    cnt_ref[...] += lax.dot_general(oh, jnp.ones((nb, 1), jnp.float32), dn,
                                    preferred_element_type=jnp.float32)


def _combine_body(ps_ref, cnt_ref, as_ref, vw_ref, vb_ref, adv_ref, gi_ref,
                  out_ref):
    nseg = ps_ref.shape[0]
    cnt = jnp.maximum(cnt_ref[...], 1.0)
    pooled = ps_ref[...] / cnt
    value = jnp.tanh(_dot(pooled, vw_ref[...]) + vb_ref[...])
    combined = value - as_ref[...] / cnt
    ids = lax.broadcasted_iota(jnp.int32, (1, nseg), 1)
    oh = (gi_ref[...] == ids).astype(jnp.float32)
    out_ref[...] = _dot(oh, combined) + adv_ref[...]


def kernel(x, edge_index, graph_indices, W1s, W1n, b1, W2s, W2n, b2, mWs, mWn,
           mb, madvW, madvb, mvalW, mvalb, bWs, bWn, bb, badvW, badvb, bvalW,
           bvalb):
    n = x.shape[0]
    e = edge_index.shape[1]
    nseg = 256
    f32 = jnp.float32

    is_maker = x[0, 2] == 1.0
    W3s = jnp.where(is_maker, mWs, bWs)
    W3n = jnp.where(is_maker, mWn, bWn)
    b3 = jnp.where(is_maker, mb, bb)
    advW = jnp.where(is_maker, madvW, badvW)
    advb = jnp.where(is_maker, madvb, badvb)
    valW = jnp.where(is_maker, mvalW, bvalW)
    valb = jnp.where(is_maker, mvalb, bvalb)

    x16 = jnp.concatenate(
        [x[:, 0:2], jnp.ones((n, 1), f32), jnp.zeros((n, 13), f32)], axis=1)

    step = NSC * NTILE * G * CH
    epad = ((e + step - 1) // step) * step
    trash = n + (jnp.arange(epad - e, dtype=jnp.int32) % 256)
    src = jnp.concatenate(
        [edge_index[0], jnp.zeros((epad - e,), jnp.int32)]).reshape(-1, CH)
    dst = jnp.concatenate([edge_index[1], trash]).reshape(-1, CH)
    nchunks = epad // CH

    w1 = jnp.concatenate([W1s, W1n], axis=0)
    w2 = jnp.concatenate([W2s, W2n], axis=0)
    w3 = jnp.concatenate([W3s, W3n], axis=0)
    b1r, b2r, b3r = b1.reshape(1, -1), b2.reshape(1, -1), b3.reshape(1, -1)
    advbr, valbr = advb.reshape(1, 1), valb.reshape(1, 1)
    gi2 = graph_indices.reshape(n, 1)

    seg1 = _sc_segsum(16, n, nchunks, core_indexed_table=False)
    seg2 = _sc_segsum(32, n, nchunks, core_indexed_table=True)

    grid = (n // BN,)
    full2 = lambda shp: pl.BlockSpec(shp, lambda i: (0, 0))
    rows2 = lambda d: pl.BlockSpec((BN, d), lambda i: (i, 0))
    rows3 = lambda d: pl.BlockSpec((NSC, BN, d), lambda i: (0, i, 0))

    src1 = src.reshape(NSC * NTILE, -1, SLAB, CH)
    dst1 = dst.reshape(NSC * NTILE, -1, SLAB, CH)
    src2 = src.reshape(NTILE, -1, SLAB, CH)
    dst2 = dst.reshape(NTILE, -1, SLAB, CH)

    p1 = seg1(x16.reshape(1, n, 16), src1, dst1).reshape(NSC, n, 16)
    h1, inv = pl.pallas_call(
        _layer1_body,
        grid=grid,
        in_specs=[rows3(16), rows2(16), full2((4, 64)), full2((1, 64))],
        out_specs=[rows3(32), rows2(1)],
        out_shape=[jax.ShapeDtypeStruct((NSC, n, 32), f32),
                   jax.ShapeDtypeStruct((n, 1), f32)],
    )(p1, x16, w1, b1r)

    s2 = seg2(h1, src2, dst2).reshape(NSC, n, 32)
    h2 = pl.pallas_call(
        _layer_body,
        grid=grid,
        in_specs=[rows3(32), rows3(32), rows2(1), full2((128, 64)),
                  full2((1, 64))],
        out_specs=rows3(32),
        out_shape=jax.ShapeDtypeStruct((NSC, n, 32), f32),
    )(h1, s2, inv, w2, b2r)

    s3 = seg2(h2, src2, dst2).reshape(NSC, n, 32)
    advn, psum, asum, cnt = pl.pallas_call(
        _head_body,
        grid=grid,
        in_specs=[rows3(32), rows3(32), rows2(1), full2((128, 64)),
                  full2((1, 64)), full2((64, 1)), full2((1, 1)), rows2(1)],
        out_specs=[rows2(1), full2((nseg, 64)), full2((nseg, 1)),
                   full2((nseg, 1))],
        out_shape=[jax.ShapeDtypeStruct((n, 1), f32),
                   jax.ShapeDtypeStruct((nseg, 64), f32),
                   jax.ShapeDtypeStruct((nseg, 1), f32),
                   jax.ShapeDtypeStruct((nseg, 1), f32)],
    )(h2, s3, inv, w3, b3r, advW, advbr, gi2)

    out = pl.pallas_call(
        _combine_body,
        grid=grid,
        in_specs=[full2((nseg, 64)), full2((nseg, 1)), full2((nseg, 1)),
                  full2((64, 1)), full2((1, 1)), rows2(1), rows2(1)],
        out_specs=rows2(1),
        out_shape=jax.ShapeDtypeStruct((n, 1), f32),
    )(psum, cnt, asum, valW, valbr, advn, gi2)
    return out[:, 0]


# speedup vs baseline: 11.1468x; 1.0669x over previous
import jax
import jax.numpy as jnp
from jax import lax
from jax.experimental import pallas as pl
from jax.experimental.pallas import tpu as pltpu
from jax.experimental.pallas import tpu_sc as plsc

NSC = 2
NTILE = 16
CH = 128
G = 14
SLAB = 28
ZB = 90
BN = 2000


def _sc_segsum(d, n, nchunks, core_indexed_table):
    na = n + 400
    assert na % (NTILE * ZB) == 0 and n % (NTILE * 625) == 0
    ptc = nchunks // (NTILE if core_indexed_table else NTILE * NSC)
    ptg = ptc // G
    slabs = n // 625

    nsl = ptc // SLAB

    def body(tbl_ref, src_ref, dst_ref, out_ref, acc, sidx, didx, rows, zbuf,
             gsem0, gsem1, gsem2, ssem0, ssem1, ssem2, isem):
        gsem = (gsem0, gsem1, gsem2)
        ssem = (ssem0, ssem1, ssem2)
        c = lax.axis_index("c")
        s = lax.axis_index("s")

        if core_indexed_table:
            tbl = tbl_ref.at[c]
            tid = s
        else:
            tbl = tbl_ref.at[0]
            tid = c * NTILE + s

        pltpu.async_copy(src_ref.at[tid, 0], sidx.at[0], isem)
        pltpu.async_copy(dst_ref.at[tid, 0], didx.at[0], isem)

        def zrow(i, _):
            for j in range(d // 16):
                zbuf[i, pl.ds(16 * j, 16)] = jnp.zeros((16,), jnp.float32)
            return 0
        lax.fori_loop(0, ZB, zrow, 0)
        zbase = s * (na // NTILE)

        def zcp(i, _):
            pltpu.sync_copy(zbuf, acc.at[pl.ds(zbase + i * ZB, ZB)])
            return 0
        lax.fori_loop(0, na // (NTILE * ZB), zcp, 0)
        plsc.subcore_barrier()

        def slab(i, _):
            p = i % 2
            pltpu.make_async_copy(src_ref.at[tid, i], sidx.at[p], isem).wait()
            pltpu.make_async_copy(dst_ref.at[tid, i], didx.at[p], isem).wait()

            @pl.when(i + 1 < nsl)
            def _():
                pltpu.async_copy(src_ref.at[tid, i + 1], sidx.at[1 - p], isem)
                pltpu.async_copy(dst_ref.at[tid, i + 1], didx.at[1 - p], isem)

            def group(g, _):
                b = g * G
                gd = {0: pltpu.async_copy(tbl.at[sidx.at[p, b]], rows.at[0],
                                          gsem[0])}
                sd = {}
                for j in range(G):
                    q = j % 3
                    if j + 1 < G:
                        r = (j + 1) % 3
                        if j + 1 - 3 in sd:
                            sd.pop(j + 1 - 3).wait()
                        gd[j + 1] = pltpu.async_copy(
                            tbl.at[sidx.at[p, b + j + 1]], rows.at[r],
                            gsem[r])
                    gd.pop(j).wait()
                    sd[j] = pltpu.async_copy(rows.at[q],
                                             acc.at[didx.at[p, b + j]],
                                             ssem[q], add=True)
                for j in sorted(sd):
                    sd.pop(j).wait()
                return 0
            lax.fori_loop(0, SLAB // G, group, 0)
            return 0
        lax.fori_loop(0, nsl, slab, 0)
        plsc.subcore_barrier()

        spt = slabs // NTILE

        def ocp(i, _):
            pltpu.sync_copy(acc.at[pl.ds((s * spt + i) * 625, 625)],
                            out_ref.at[c, s * spt + i])
            return 0
        lax.fori_loop(0, spt, ocp, 0)

    mesh = plsc.VectorSubcoreMesh(core_axis_name="c", subcore_axis_name="s")
    return pl.kernel(
        body,
        out_type=jax.ShapeDtypeStruct((NSC, slabs, 625, d), jnp.float32),
        mesh=mesh,
        compiler_params=pltpu.CompilerParams(use_tc_tiling_on_sc=False),
        scratch_types=[
            pltpu.VMEM_SHARED((na, d), jnp.float32),
            pltpu.VMEM((2, SLAB, CH), jnp.int32),
            pltpu.VMEM((2, SLAB, CH), jnp.int32),
            pltpu.VMEM((3, CH, d), jnp.float32),
            pltpu.VMEM((ZB, d), jnp.float32),
            pltpu.SemaphoreType.DMA,
            pltpu.SemaphoreType.DMA,
            pltpu.SemaphoreType.DMA,
            pltpu.SemaphoreType.DMA,
            pltpu.SemaphoreType.DMA,
            pltpu.SemaphoreType.DMA,
            pltpu.SemaphoreType.DMA,
        ],
    )


def _dot(a, b):
    return jnp.dot(a, b, preferred_element_type=jnp.float32)


def _layer1_body(p_ref, x_ref, w_ref, b_ref, h_ref, inv_ref):
    p = p_ref[0] + p_ref[1]
    inv = 1.0 / jnp.maximum(p[:, 2:3], 1.0)
    inp = jnp.concatenate([x_ref[:, 0:2], p[:, 0:2] * inv], axis=1)
    h = jnp.maximum(_dot(inp, w_ref[...]) + b_ref[...], 0.0)
    h_ref[0] = h[:, 0:32]
    h_ref[1] = h[:, 32:64]
    inv_ref[...] = inv


def _layer_body(h_ref, s_ref, inv_ref, w_ref, b_ref, o_ref):
    h = jnp.concatenate([h_ref[0], h_ref[1]], axis=1)
    agg = jnp.concatenate([s_ref[0], s_ref[1]], axis=1) * inv_ref[...]
    o = jnp.maximum(_dot(jnp.concatenate([h, agg], axis=1), w_ref[...])
                    + b_ref[...], 0.0)
    o_ref[0] = o[:, 0:32]
    o_ref[1] = o[:, 32:64]


def _head_body(h_ref, s_ref, inv_ref, w_ref, b_ref, aw_ref, ab_ref, gi_ref,
               adv_ref, ps_ref, as_ref, cnt_ref):
    nb = gi_ref.shape[0]
    h = jnp.concatenate([h_ref[0], h_ref[1]], axis=1)
    agg = jnp.concatenate([s_ref[0], s_ref[1]], axis=1) * inv_ref[...]
    hh = jnp.maximum(_dot(jnp.concatenate([h, agg], axis=1), w_ref[...])
                     + b_ref[...], 0.0)
    advn = 2.0 * jnp.tanh(_dot(hh, aw_ref[...]) + ab_ref[...])
    adv_ref[...] = advn

    nseg = ps_ref.shape[0]
    ids = lax.broadcasted_iota(jnp.int32, (1, nseg), 1)
    oh = (gi_ref[...] == ids).astype(jnp.float32)

    @pl.when(pl.program_id(0) == 0)
    def _():
        ps_ref[...] = jnp.zeros_like(ps_ref)
        as_ref[...] = jnp.zeros_like(as_ref)
        cnt_ref[...] = jnp.zeros_like(cnt_ref)

    dn = (((0,), (0,)), ((), ()))
    ps_ref[...] += lax.dot_general(oh, hh, dn,
                                   preferred_element_type=jnp.float32)
    as_ref[...] += lax.dot_general(oh, advn, dn,
                                   preferred_element_type=jnp.float32)
    cnt_ref[...] += lax.dot_general(oh, jnp.ones((nb, 1), jnp.float32), dn,
                                    preferred_element_type=jnp.float32)


def _combine_body(ps_ref, cnt_ref, as_ref, vw_ref, vb_ref, adv_ref, gi_ref,
                  out_ref):
    nseg = ps_ref.shape[0]
    cnt = jnp.maximum(cnt_ref[...], 1.0)
    pooled = ps_ref[...] / cnt
    value = jnp.tanh(_dot(pooled, vw_ref[...]) + vb_ref[...])
    combined = value - as_ref[...] / cnt
    ids = lax.broadcasted_iota(jnp.int32, (1, nseg), 1)
    oh = (gi_ref[...] == ids).astype(jnp.float32)
    out_ref[...] = _dot(oh, combined) + adv_ref[...]


def kernel(x, edge_index, graph_indices, W1s, W1n, b1, W2s, W2n, b2, mWs, mWn,
           mb, madvW, madvb, mvalW, mvalb, bWs, bWn, bb, badvW, badvb, bvalW,
           bvalb):
    n = x.shape[0]
    e = edge_index.shape[1]
    nseg = 256
    f32 = jnp.float32

    is_maker = x[0, 2] == 1.0
    W3s = jnp.where(is_maker, mWs, bWs)
    W3n = jnp.where(is_maker, mWn, bWn)
    b3 = jnp.where(is_maker, mb, bb)
    advW = jnp.where(is_maker, madvW, badvW)
    advb = jnp.where(is_maker, madvb, badvb)
    valW = jnp.where(is_maker, mvalW, bvalW)
    valb = jnp.where(is_maker, mvalb, bvalb)

    x16 = jnp.concatenate(
        [x[:, 0:2], jnp.ones((n, 1), f32), jnp.zeros((n, 13), f32)], axis=1)

    step = NSC * NTILE * G * CH
    epad = ((e + step - 1) // step) * step
    trash = n + (jnp.arange(epad - e, dtype=jnp.int32) % 256)
    src = jnp.concatenate(
        [edge_index[0], jnp.zeros((epad - e,), jnp.int32)]).reshape(-1, CH)
    dst = jnp.concatenate([edge_index[1], trash]).reshape(-1, CH)
    nchunks = epad // CH

    w1 = jnp.concatenate([W1s, W1n], axis=0)
    w2 = jnp.concatenate([W2s, W2n], axis=0)
    w3 = jnp.concatenate([W3s, W3n], axis=0)
    b1r, b2r, b3r = b1.reshape(1, -1), b2.reshape(1, -1), b3.reshape(1, -1)
    advbr, valbr = advb.reshape(1, 1), valb.reshape(1, 1)
    gi2 = graph_indices.reshape(n, 1)

    seg1 = _sc_segsum(16, n, nchunks, core_indexed_table=False)
    seg2 = _sc_segsum(32, n, nchunks, core_indexed_table=True)

    grid = (n // BN,)
    full2 = lambda shp: pl.BlockSpec(shp, lambda i: (0, 0))
    rows2 = lambda d: pl.BlockSpec((BN, d), lambda i: (i, 0))
    rows3 = lambda d: pl.BlockSpec((NSC, BN, d), lambda i: (0, i, 0))

    src1 = src.reshape(NSC * NTILE, -1, SLAB, CH)
    dst1 = dst.reshape(NSC * NTILE, -1, SLAB, CH)
    src2 = src.reshape(NTILE, -1, SLAB, CH)
    dst2 = dst.reshape(NTILE, -1, SLAB, CH)

    p1 = seg1(x16.reshape(1, n, 16), src1, dst1).reshape(NSC, n, 16)
    h1, inv = pl.pallas_call(
        _layer1_body,
        grid=grid,
        in_specs=[rows3(16), rows2(16), full2((4, 64)), full2((1, 64))],
        out_specs=[rows3(32), rows2(1)],
        out_shape=[jax.ShapeDtypeStruct((NSC, n, 32), f32),
                   jax.ShapeDtypeStruct((n, 1), f32)],
    )(p1, x16, w1, b1r)

    s2 = seg2(h1, src2, dst2).reshape(NSC, n, 32)
    h2 = pl.pallas_call(
        _layer_body,
        grid=grid,
        in_specs=[rows3(32), rows3(32), rows2(1), full2((128, 64)),
                  full2((1, 64))],
        out_specs=rows3(32),
        out_shape=jax.ShapeDtypeStruct((NSC, n, 32), f32),
    )(h1, s2, inv, w2, b2r)

    s3 = seg2(h2, src2, dst2).reshape(NSC, n, 32)
    advn, psum, asum, cnt = pl.pallas_call(
        _head_body,
        grid=grid,
        in_specs=[rows3(32), rows3(32), rows2(1), full2((128, 64)),
                  full2((1, 64)), full2((64, 1)), full2((1, 1)), rows2(1)],
        out_specs=[rows2(1), full2((nseg, 64)), full2((nseg, 1)),
                   full2((nseg, 1))],
        out_shape=[jax.ShapeDtypeStruct((n, 1), f32),
                   jax.ShapeDtypeStruct((nseg, 64), f32),
                   jax.ShapeDtypeStruct((nseg, 1), f32),
                   jax.ShapeDtypeStruct((nseg, 1), f32)],
    )(h2, s3, inv, w3, b3r, advW, advbr, gi2)

    out = pl.pallas_call(
        _combine_body,
        grid=grid,
        in_specs=[full2((nseg, 64)), full2((nseg, 1)), full2((nseg, 1)),
                  full2((64, 1)), full2((1, 1)), rows2(1), rows2(1)],
        out_specs=rows2(1),
        out_shape=jax.ShapeDtypeStruct((n, 1), f32),
    )(psum, cnt, asum, valW, valbr, advn, gi2)
    return out[:, 0]


# speedup vs baseline: 11.5102x; 1.0326x over previous
import jax
import jax.numpy as jnp
from jax import lax
from jax.experimental import pallas as pl
from jax.experimental.pallas import tpu as pltpu
from jax.experimental.pallas import tpu_sc as plsc

NSC = 2
NTILE = 16
CH = 128
KC = 2
SLAB = 14
ZB = 90
BN = 2000


def _sc_segsum(d, n, nchunks, core_indexed_table):
    na = n + 400
    assert na % (NTILE * ZB) == 0 and n % (NTILE * 625) == 0
    ptc = nchunks // (NTILE if core_indexed_table else NTILE * NSC)
    slabs = n // 625

    nsl = ptc // SLAB

    def body(tbl_ref, src_ref, dst_ref, out_ref, acc, sidx, didx, rows, zbuf,
             gsem0, gsem1, ssem0, ssem1, isem):
        gsem = (gsem0, gsem1)
        ssem = (ssem0, ssem1)
        c = lax.axis_index("c")
        s = lax.axis_index("s")

        if core_indexed_table:
            tbl = tbl_ref.at[c]
            tid = s
        else:
            tbl = tbl_ref.at[0]
            tid = c * NTILE + s

        pltpu.async_copy(src_ref.at[tid, 0], sidx.at[0], isem)
        pltpu.async_copy(dst_ref.at[tid, 0], didx.at[0], isem)

        def zrow(i, _):
            for j in range(d // 16):
                zbuf[i, pl.ds(16 * j, 16)] = jnp.zeros((16,), jnp.float32)
            return 0
        lax.fori_loop(0, ZB, zrow, 0)
        zbase = s * (na // NTILE)

        def zcp(i, _):
            pltpu.sync_copy(zbuf, acc.at[pl.ds(zbase + i * ZB, ZB)])
            return 0
        lax.fori_loop(0, na // (NTILE * ZB), zcp, 0)
        plsc.subcore_barrier()

        def slab(i, _):
            p = i % 2
            pltpu.make_async_copy(src_ref.at[tid, i], sidx.at[p], isem).wait()
            pltpu.make_async_copy(dst_ref.at[tid, i], didx.at[p], isem).wait()

            @pl.when(i + 1 < nsl)
            def _():
                pltpu.async_copy(src_ref.at[tid, i + 1], sidx.at[1 - p], isem)
                pltpu.async_copy(dst_ref.at[tid, i + 1], didx.at[1 - p], isem)

            U = SLAB // KC
            gd = {0: pltpu.async_copy(tbl.at[sidx.at[p, 0]],
                                      rows.at[0], gsem[0])}
            sd = {}
            for u in range(U):
                q = u % 2
                if u + 1 < U:
                    if u - 1 in sd:
                        for w in sd.pop(u - 1):
                            w.wait()
                    gd[u + 1] = pltpu.async_copy(
                        tbl.at[sidx.at[p, u + 1]],
                        rows.at[1 - q], gsem[1 - q])
                gd.pop(u).wait()
                sd[u] = [pltpu.async_copy(
                    rows.at[q, pl.ds(k * CH, CH)],
                    acc.at[didx.at[p, KC * u + k]],
                    ssem[q], add=True) for k in range(KC)]
            for u in sorted(sd):
                for w in sd.pop(u):
                    w.wait()
            return 0
        lax.fori_loop(0, nsl, slab, 0)
        plsc.subcore_barrier()

        spt = slabs // NTILE

        def ocp(i, _):
            pltpu.sync_copy(acc.at[pl.ds((s * spt + i) * 625, 625)],
                            out_ref.at[c, s * spt + i])
            return 0
        lax.fori_loop(0, spt, ocp, 0)

    mesh = plsc.VectorSubcoreMesh(core_axis_name="c", subcore_axis_name="s")
    return pl.kernel(
        body,
        out_type=jax.ShapeDtypeStruct((NSC, slabs, 625, d), jnp.float32),
        mesh=mesh,
        compiler_params=pltpu.CompilerParams(use_tc_tiling_on_sc=False),
        scratch_types=[
            pltpu.VMEM_SHARED((na, d), jnp.float32),
            pltpu.VMEM((2, SLAB // KC, KC * CH), jnp.int32),
            pltpu.VMEM((2, SLAB, CH), jnp.int32),
            pltpu.VMEM((2, KC * CH, d), jnp.float32),
            pltpu.VMEM((ZB, d), jnp.float32),
            pltpu.SemaphoreType.DMA,
            pltpu.SemaphoreType.DMA,
            pltpu.SemaphoreType.DMA,
            pltpu.SemaphoreType.DMA,
            pltpu.SemaphoreType.DMA,
        ],
    )


def _dot(a, b):
    return jnp.dot(a, b, preferred_element_type=jnp.float32)


def _layer1_body(p_ref, x_ref, w_ref, b_ref, h_ref, inv_ref):
    p = p_ref[0] + p_ref[1]
    inv = 1.0 / jnp.maximum(p[:, 2:3], 1.0)
    inp = jnp.concatenate([x_ref[:, 0:2], p[:, 0:2] * inv], axis=1)
    h = jnp.maximum(_dot(inp, w_ref[...]) + b_ref[...], 0.0)
    h_ref[0] = h[:, 0:32]
    h_ref[1] = h[:, 32:64]
    inv_ref[...] = inv


def _layer_body(h_ref, s_ref, inv_ref, w_ref, b_ref, o_ref):
    h = jnp.concatenate([h_ref[0], h_ref[1]], axis=1)
    agg = jnp.concatenate([s_ref[0], s_ref[1]], axis=1) * inv_ref[...]
    o = jnp.maximum(_dot(jnp.concatenate([h, agg], axis=1), w_ref[...])
                    + b_ref[...], 0.0)
    o_ref[0] = o[:, 0:32]
    o_ref[1] = o[:, 32:64]


def _head_body(h_ref, s_ref, inv_ref, w_ref, b_ref, aw_ref, ab_ref, gi_ref,
               adv_ref, ps_ref, as_ref, cnt_ref):
    nb = gi_ref.shape[0]
    h = jnp.concatenate([h_ref[0], h_ref[1]], axis=1)
    agg = jnp.concatenate([s_ref[0], s_ref[1]], axis=1) * inv_ref[...]
    hh = jnp.maximum(_dot(jnp.concatenate([h, agg], axis=1), w_ref[...])
                     + b_ref[...], 0.0)
    advn = 2.0 * jnp.tanh(_dot(hh, aw_ref[...]) + ab_ref[...])
    adv_ref[...] = advn

    nseg = ps_ref.shape[0]
    ids = lax.broadcasted_iota(jnp.int32, (1, nseg), 1)
    oh = (gi_ref[...] == ids).astype(jnp.float32)

    @pl.when(pl.program_id(0) == 0)
    def _():
        ps_ref[...] = jnp.zeros_like(ps_ref)
        as_ref[...] = jnp.zeros_like(as_ref)
        cnt_ref[...] = jnp.zeros_like(cnt_ref)

    dn = (((0,), (0,)), ((), ()))
    ps_ref[...] += lax.dot_general(oh, hh, dn,
                                   preferred_element_type=jnp.float32)
    as_ref[...] += lax.dot_general(oh, advn, dn,
                                   preferred_element_type=jnp.float32)
    cnt_ref[...] += lax.dot_general(oh, jnp.ones((nb, 1), jnp.float32), dn,
                                    preferred_element_type=jnp.float32)


def _combine_body(ps_ref, cnt_ref, as_ref, vw_ref, vb_ref, adv_ref, gi_ref,
                  out_ref):
    nseg = ps_ref.shape[0]
    cnt = jnp.maximum(cnt_ref[...], 1.0)
    pooled = ps_ref[...] / cnt
    value = jnp.tanh(_dot(pooled, vw_ref[...]) + vb_ref[...])
    combined = value - as_ref[...] / cnt
    ids = lax.broadcasted_iota(jnp.int32, (1, nseg), 1)
    oh = (gi_ref[...] == ids).astype(jnp.float32)
    out_ref[...] = _dot(oh, combined) + adv_ref[...]


def kernel(x, edge_index, graph_indices, W1s, W1n, b1, W2s, W2n, b2, mWs, mWn,
           mb, madvW, madvb, mvalW, mvalb, bWs, bWn, bb, badvW, badvb, bvalW,
           bvalb):
    n = x.shape[0]
    e = edge_index.shape[1]
    nseg = 256
    f32 = jnp.float32

    is_maker = x[0, 2] == 1.0
    W3s = jnp.where(is_maker, mWs, bWs)
    W3n = jnp.where(is_maker, mWn, bWn)
    b3 = jnp.where(is_maker, mb, bb)
    advW = jnp.where(is_maker, madvW, badvW)
    advb = jnp.where(is_maker, madvb, badvb)
    valW = jnp.where(is_maker, mvalW, bvalW)
    valb = jnp.where(is_maker, mvalb, bvalb)

    x16 = jnp.concatenate(
        [x[:, 0:2], jnp.ones((n, 1), f32), jnp.zeros((n, 13), f32)], axis=1)

    step = NSC * NTILE * SLAB * CH
    epad = ((e + step - 1) // step) * step
    trash = n + (jnp.arange(epad - e, dtype=jnp.int32) % 256)
    src = jnp.concatenate(
        [edge_index[0], jnp.zeros((epad - e,), jnp.int32)]).reshape(-1, CH)
    dst = jnp.concatenate([edge_index[1], trash]).reshape(-1, CH)
    nchunks = epad // CH

    w1 = jnp.concatenate([W1s, W1n], axis=0)
    w2 = jnp.concatenate([W2s, W2n], axis=0)
    w3 = jnp.concatenate([W3s, W3n], axis=0)
    b1r, b2r, b3r = b1.reshape(1, -1), b2.reshape(1, -1), b3.reshape(1, -1)
    advbr, valbr = advb.reshape(1, 1), valb.reshape(1, 1)
    gi2 = graph_indices.reshape(n, 1)

    seg1 = _sc_segsum(16, n, nchunks, core_indexed_table=False)
    seg2 = _sc_segsum(32, n, nchunks, core_indexed_table=True)

    grid = (n // BN,)
    full2 = lambda shp: pl.BlockSpec(shp, lambda i: (0, 0))
    rows2 = lambda d: pl.BlockSpec((BN, d), lambda i: (i, 0))
    rows3 = lambda d: pl.BlockSpec((NSC, BN, d), lambda i: (0, i, 0))

    src1 = src.reshape(NSC * NTILE, -1, SLAB // KC, KC * CH)
    dst1 = dst.reshape(NSC * NTILE, -1, SLAB, CH)
    src2 = src.reshape(NTILE, -1, SLAB // KC, KC * CH)
    dst2 = dst.reshape(NTILE, -1, SLAB, CH)

    p1 = seg1(x16.reshape(1, n, 16), src1, dst1).reshape(NSC, n, 16)
    h1, inv = pl.pallas_call(
        _layer1_body,
        grid=grid,
        in_specs=[rows3(16), rows2(16), full2((4, 64)), full2((1, 64))],
        out_specs=[rows3(32), rows2(1)],
        out_shape=[jax.ShapeDtypeStruct((NSC, n, 32), f32),
                   jax.ShapeDtypeStruct((n, 1), f32)],
    )(p1, x16, w1, b1r)

    s2 = seg2(h1, src2, dst2).reshape(NSC, n, 32)
    h2 = pl.pallas_call(
        _layer_body,
        grid=grid,
        in_specs=[rows3(32), rows3(32), rows2(1), full2((128, 64)),
                  full2((1, 64))],
        out_specs=rows3(32),
        out_shape=jax.ShapeDtypeStruct((NSC, n, 32), f32),
    )(h1, s2, inv, w2, b2r)

    s3 = seg2(h2, src2, dst2).reshape(NSC, n, 32)
    advn, psum, asum, cnt = pl.pallas_call(
        _head_body,
        grid=grid,
        in_specs=[rows3(32), rows3(32), rows2(1), full2((128, 64)),
                  full2((1, 64)), full2((64, 1)), full2((1, 1)), rows2(1)],
        out_specs=[rows2(1), full2((nseg, 64)), full2((nseg, 1)),
                   full2((nseg, 1))],
        out_shape=[jax.ShapeDtypeStruct((n, 1), f32),
                   jax.ShapeDtypeStruct((nseg, 64), f32),
                   jax.ShapeDtypeStruct((nseg, 1), f32),
                   jax.ShapeDtypeStruct((nseg, 1), f32)],
    )(h2, s3, inv, w3, b3r, advW, advbr, gi2)

    out = pl.pallas_call(
        _combine_body,
        grid=grid,
        in_specs=[full2((nseg, 64)), full2((nseg, 1)), full2((nseg, 1)),
                  full2((64, 1)), full2((1, 1)), rows2(1), rows2(1)],
        out_specs=rows2(1),
        out_shape=jax.ShapeDtypeStruct((n, 1), f32),
    )(psum, cnt, asum, valW, valbr, advn, gi2)
    return out[:, 0]


# speedup vs baseline: 11.5607x; 1.0044x over previous
import jax
import jax.numpy as jnp
from jax import lax
from jax.experimental import pallas as pl
from jax.experimental.pallas import tpu as pltpu
from jax.experimental.pallas import tpu_sc as plsc

NSC = 2
NTILE = 16
CH = 128
KC = 2
SLAB = 14
ZB = 90
BN = 2000


def _sc_segsum(d, n, nchunks, core_indexed_table):
    na = n + 400
    assert na % (NTILE * ZB) == 0 and n % (NTILE * 625) == 0
    ptc = nchunks // (NTILE if core_indexed_table else NTILE * NSC)
    slabs = n // 625

    nsl = ptc // SLAB

    def body(tbl_ref, src_ref, dst_ref, out_ref, acc, sidx, didx, rows, zbuf,
             gsem0, gsem1, ssem0, ssem1, isem):
        gsem = (gsem0, gsem1)
        ssem = (ssem0, ssem1)
        c = lax.axis_index("c")
        s = lax.axis_index("s")

        if core_indexed_table:
            tbl = tbl_ref.at[c]
            tid = s
        else:
            tbl = tbl_ref.at[0]
            tid = c * NTILE + s

        pltpu.async_copy(src_ref.at[tid, 0], sidx.at[0], isem)
        pltpu.async_copy(dst_ref.at[tid, 0], didx.at[0], isem)

        def zrow(i, _):
            for j in range(d // 16):
                zbuf[i, pl.ds(16 * j, 16)] = jnp.zeros((16,), jnp.float32)
            return 0
        lax.fori_loop(0, ZB, zrow, 0)
        zbase = s * (na // NTILE)

        def zcp(i, _):
            pltpu.async_copy(zbuf, acc.at[pl.ds(zbase + i * ZB, ZB)], ssem0)
            return 0
        lax.fori_loop(0, na // (NTILE * ZB), zcp, 0)

        def zwait(i, _):
            pltpu.make_async_copy(
                zbuf, acc.at[pl.ds(zbase + i * ZB, ZB)], ssem0).wait()
            return 0
        lax.fori_loop(0, na // (NTILE * ZB), zwait, 0)
        plsc.subcore_barrier()

        def slab(i, _):
            p = i % 2
            pltpu.make_async_copy(src_ref.at[tid, i], sidx.at[p], isem).wait()
            pltpu.make_async_copy(dst_ref.at[tid, i], didx.at[p], isem).wait()

            @pl.when(i + 1 < nsl)
            def _():
                pltpu.async_copy(src_ref.at[tid, i + 1], sidx.at[1 - p], isem)
                pltpu.async_copy(dst_ref.at[tid, i + 1], didx.at[1 - p], isem)

            U = SLAB // KC
            gd = {0: pltpu.async_copy(tbl.at[sidx.at[p, 0]],
                                      rows.at[0], gsem[0])}
            sd = {}
            for u in range(U):
                q = u % 2
                if u + 1 < U:
                    if u - 1 in sd:
                        for w in sd.pop(u - 1):
                            w.wait()
                    gd[u + 1] = pltpu.async_copy(
                        tbl.at[sidx.at[p, u + 1]],
                        rows.at[1 - q], gsem[1 - q])
                gd.pop(u).wait()
                sd[u] = [pltpu.async_copy(
                    rows.at[q, pl.ds(k * CH, CH)],
                    acc.at[didx.at[p, KC * u + k]],
                    ssem[q], add=True) for k in range(KC)]
            for u in sorted(sd):
                for w in sd.pop(u):
                    w.wait()
            return 0
        lax.fori_loop(0, nsl, slab, 0)
        plsc.subcore_barrier()

        spt = slabs // NTILE

        def ocp(i, _):
            pltpu.sync_copy(acc.at[pl.ds((s * spt + i) * 625, 625)],
                            out_ref.at[c, s * spt + i])
            return 0
        lax.fori_loop(0, spt, ocp, 0)

    mesh = plsc.VectorSubcoreMesh(core_axis_name="c", subcore_axis_name="s")
    return pl.kernel(
        body,
        out_type=jax.ShapeDtypeStruct((NSC, slabs, 625, d), jnp.float32),
        mesh=mesh,
        compiler_params=pltpu.CompilerParams(use_tc_tiling_on_sc=False),
        scratch_types=[
            pltpu.VMEM_SHARED((na, d), jnp.float32),
            pltpu.VMEM((2, SLAB // KC, KC * CH), jnp.int32),
            pltpu.VMEM((2, SLAB, CH), jnp.int32),
            pltpu.VMEM((2, KC * CH, d), jnp.float32),
            pltpu.VMEM((ZB, d), jnp.float32),
            pltpu.SemaphoreType.DMA,
            pltpu.SemaphoreType.DMA,
            pltpu.SemaphoreType.DMA,
            pltpu.SemaphoreType.DMA,
            pltpu.SemaphoreType.DMA,
        ],
    )


def _dot(a, b):
    return jnp.dot(a, b, preferred_element_type=jnp.float32)


def _layer1_body(p_ref, x_ref, w_ref, b_ref, h_ref, inv_ref):
    p = p_ref[0] + p_ref[1]
    inv = 1.0 / jnp.maximum(p[:, 2:3], 1.0)
    inp = jnp.concatenate([x_ref[:, 0:2], p[:, 0:2] * inv], axis=1)
    h = jnp.maximum(_dot(inp, w_ref[...]) + b_ref[...], 0.0)
    h_ref[0] = h[:, 0:32]
    h_ref[1] = h[:, 32:64]
    inv_ref[...] = inv


def _layer_body(h_ref, s_ref, inv_ref, w_ref, b_ref, o_ref):
    h = jnp.concatenate([h_ref[0], h_ref[1]], axis=1)
    agg = jnp.concatenate([s_ref[0], s_ref[1]], axis=1) * inv_ref[...]
    o = jnp.maximum(_dot(jnp.concatenate([h, agg], axis=1), w_ref[...])
                    + b_ref[...], 0.0)
    o_ref[0] = o[:, 0:32]
    o_ref[1] = o[:, 32:64]


def _head_body(h_ref, s_ref, inv_ref, w_ref, b_ref, aw_ref, ab_ref, gi_ref,
               adv_ref, ps_ref, as_ref, cnt_ref):
    nb = gi_ref.shape[0]
    h = jnp.concatenate([h_ref[0], h_ref[1]], axis=1)
    agg = jnp.concatenate([s_ref[0], s_ref[1]], axis=1) * inv_ref[...]
    hh = jnp.maximum(_dot(jnp.concatenate([h, agg], axis=1), w_ref[...])
                     + b_ref[...], 0.0)
    advn = 2.0 * jnp.tanh(_dot(hh, aw_ref[...]) + ab_ref[...])
    adv_ref[...] = advn

    nseg = ps_ref.shape[0]
    ids = lax.broadcasted_iota(jnp.int32, (1, nseg), 1)
    oh = (gi_ref[...] == ids).astype(jnp.float32)

    @pl.when(pl.program_id(0) == 0)
    def _():
        ps_ref[...] = jnp.zeros_like(ps_ref)
        as_ref[...] = jnp.zeros_like(as_ref)
        cnt_ref[...] = jnp.zeros_like(cnt_ref)

    dn = (((0,), (0,)), ((), ()))
    ps_ref[...] += lax.dot_general(oh, hh, dn,
                                   preferred_element_type=jnp.float32)
    as_ref[...] += lax.dot_general(oh, advn, dn,
                                   preferred_element_type=jnp.float32)
    cnt_ref[...] += lax.dot_general(oh, jnp.ones((nb, 1), jnp.float32), dn,
                                    preferred_element_type=jnp.float32)


def _combine_body(ps_ref, cnt_ref, as_ref, vw_ref, vb_ref, adv_ref, gi_ref,
                  out_ref):
    nseg = ps_ref.shape[0]
    cnt = jnp.maximum(cnt_ref[...], 1.0)
    pooled = ps_ref[...] / cnt
    value = jnp.tanh(_dot(pooled, vw_ref[...]) + vb_ref[...])
    combined = value - as_ref[...] / cnt
    ids = lax.broadcasted_iota(jnp.int32, (1, nseg), 1)
    oh = (gi_ref[...] == ids).astype(jnp.float32)
    out_ref[...] = _dot(oh, combined) + adv_ref[...]


def kernel(x, edge_index, graph_indices, W1s, W1n, b1, W2s, W2n, b2, mWs, mWn,
           mb, madvW, madvb, mvalW, mvalb, bWs, bWn, bb, badvW, badvb, bvalW,
           bvalb):
    n = x.shape[0]
    e = edge_index.shape[1]
    nseg = 256
    f32 = jnp.float32

    is_maker = x[0, 2] == 1.0
    W3s = jnp.where(is_maker, mWs, bWs)
    W3n = jnp.where(is_maker, mWn, bWn)
    b3 = jnp.where(is_maker, mb, bb)
    advW = jnp.where(is_maker, madvW, badvW)
    advb = jnp.where(is_maker, madvb, badvb)
    valW = jnp.where(is_maker, mvalW, bvalW)
    valb = jnp.where(is_maker, mvalb, bvalb)

    x16 = jnp.concatenate(
        [x[:, 0:2], jnp.ones((n, 1), f32), jnp.zeros((n, 13), f32)], axis=1)

    step = NSC * NTILE * SLAB * CH
    epad = ((e + step - 1) // step) * step
    trash = n + (jnp.arange(epad - e, dtype=jnp.int32) % 256)
    src = jnp.concatenate(
        [edge_index[0], jnp.zeros((epad - e,), jnp.int32)]).reshape(-1, CH)
    dst = jnp.concatenate([edge_index[1], trash]).reshape(-1, CH)
    nchunks = epad // CH

    w1 = jnp.concatenate([W1s, W1n], axis=0)
    w2 = jnp.concatenate([W2s, W2n], axis=0)
    w3 = jnp.concatenate([W3s, W3n], axis=0)
    b1r, b2r, b3r = b1.reshape(1, -1), b2.reshape(1, -1), b3.reshape(1, -1)
    advbr, valbr = advb.reshape(1, 1), valb.reshape(1, 1)
    gi2 = graph_indices.reshape(n, 1)

    seg1 = _sc_segsum(16, n, nchunks, core_indexed_table=False)
    seg2 = _sc_segsum(32, n, nchunks, core_indexed_table=True)

    grid = (n // BN,)
    full2 = lambda shp: pl.BlockSpec(shp, lambda i: (0, 0))
    rows2 = lambda d: pl.BlockSpec((BN, d), lambda i: (i, 0))
    rows3 = lambda d: pl.BlockSpec((NSC, BN, d), lambda i: (0, i, 0))

    src1 = src.reshape(NSC * NTILE, -1, SLAB // KC, KC * CH)
    dst1 = dst.reshape(NSC * NTILE, -1, SLAB, CH)
    src2 = src.reshape(NTILE, -1, SLAB // KC, KC * CH)
    dst2 = dst.reshape(NTILE, -1, SLAB, CH)

    p1 = seg1(x16.reshape(1, n, 16), src1, dst1).reshape(NSC, n, 16)
    h1, inv = pl.pallas_call(
        _layer1_body,
        grid=grid,
        in_specs=[rows3(16), rows2(16), full2((4, 64)), full2((1, 64))],
        out_specs=[rows3(32), rows2(1)],
        out_shape=[jax.ShapeDtypeStruct((NSC, n, 32), f32),
                   jax.ShapeDtypeStruct((n, 1), f32)],
    )(p1, x16, w1, b1r)

    s2 = seg2(h1, src2, dst2).reshape(NSC, n, 32)
    h2 = pl.pallas_call(
        _layer_body,
        grid=grid,
        in_specs=[rows3(32), rows3(32), rows2(1), full2((128, 64)),
                  full2((1, 64))],
        out_specs=rows3(32),
        out_shape=jax.ShapeDtypeStruct((NSC, n, 32), f32),
    )(h1, s2, inv, w2, b2r)

    s3 = seg2(h2, src2, dst2).reshape(NSC, n, 32)
    advn, psum, asum, cnt = pl.pallas_call(
        _head_body,
        grid=grid,
        in_specs=[rows3(32), rows3(32), rows2(1), full2((128, 64)),
                  full2((1, 64)), full2((64, 1)), full2((1, 1)), rows2(1)],
        out_specs=[rows2(1), full2((nseg, 64)), full2((nseg, 1)),
                   full2((nseg, 1))],
        out_shape=[jax.ShapeDtypeStruct((n, 1), f32),
                   jax.ShapeDtypeStruct((nseg, 64), f32),
                   jax.ShapeDtypeStruct((nseg, 1), f32),
                   jax.ShapeDtypeStruct((nseg, 1), f32)],
    )(h2, s3, inv, w3, b3r, advW, advbr, gi2)

    out = pl.pallas_call(
        _combine_body,
        grid=grid,
        in_specs=[full2((nseg, 64)), full2((nseg, 1)), full2((nseg, 1)),
                  full2((64, 1)), full2((1, 1)), rows2(1), rows2(1)],
        out_specs=rows2(1),
        out_shape=jax.ShapeDtypeStruct((n, 1), f32),
    )(psum, cnt, asum, valW, valbr, advn, gi2)
    return out[:, 0]
